# Initial kernel scaffold; baseline (speedup 1.0000x reference)
import jax
import jax.numpy as jnp
from jax.experimental import pallas as pl


def kernel(x, edge_index, edge_weight, W, bias):
    raise NotImplementedError("write your pallas kernel here")


# speedup vs baseline: 3.9806x; 3.9806x over previous
import dataclasses
import functools

import jax
import jax.numpy as jnp
from jax import lax
from jax.experimental import pallas as pl
from jax.experimental.pallas import tpu as pltpu
from jax.experimental.pallas import tpu_sc as plsc

NC = 2
NS = 16
LANES = 16
GK = 128


def _sc_aggregate(x, src, dst, ew, n_pad, d, n_chunks):
    rows_per_tile = n_pad // NS
    assert rows_per_tile % 8 == 0
    mesh = plsc.VectorSubcoreMesh(core_axis_name="c", subcore_axis_name="s")
    cp = pltpu.CompilerParams()
    if "needs_layout_passes" in pltpu.CompilerParams.__dataclass_fields__:
        cp = dataclasses.replace(cp, needs_layout_passes=False)

    @functools.partial(
        pl.kernel,
        mesh=mesh,
        compiler_params=cp,
        out_type=jax.ShapeDtypeStruct((NC, n_pad, d), jnp.float32),
        scratch_types=[
            pltpu.VMEM_SHARED((n_pad, d), jnp.float32),
            pltpu.VMEM((n_chunks, GK), jnp.int32),
            pltpu.VMEM((n_chunks, GK), jnp.int32),
            pltpu.VMEM((n_chunks, GK), jnp.float32),
            pltpu.VMEM((GK, d), jnp.float32),
        ],
    )
    def agg_kernel(x_hbm, src_hbm, dst_hbm, ew_hbm, part_hbm,
                   acc_sh, src_v, dst_v, ew_v, rows_v):
        c = lax.axis_index("c")
        s = lax.axis_index("s")
        gwid = c * NS + s

        @pl.loop(0, GK)
        def _(r):
            for ch in range(d // LANES):
                rows_v[r, pl.ds(ch * LANES, LANES)] = jnp.zeros((LANES,), jnp.float32)

        base = s * rows_per_tile
        for k in range(rows_per_tile // GK):
            pltpu.sync_copy(rows_v, acc_sh.at[pl.ds(base + k * GK, GK)])
        rem = rows_per_tile % GK
        if rem:
            pltpu.sync_copy(rows_v.at[pl.ds(0, rem)],
                            acc_sh.at[pl.ds(base + rows_per_tile - rem, rem)])
        plsc.subcore_barrier()

        pltpu.sync_copy(src_hbm.at[gwid], src_v)
        pltpu.sync_copy(dst_hbm.at[gwid], dst_v)
        pltpu.sync_copy(ew_hbm.at[gwid], ew_v)

        @pl.loop(0, n_chunks)
        def _(i):
            pltpu.sync_copy(x_hbm.at[src_v.at[i]], rows_v)

            @pl.loop(0, GK)
            def _(e):
                wvec = plsc.load_gather(
                    ew_v, [jnp.full((LANES,), i, jnp.int32),
                           jnp.full((LANES,), e, jnp.int32)])
                for ch in range(d // LANES):
                    sl = pl.ds(ch * LANES, LANES)
                    rows_v[e, sl] = rows_v[e, sl] * wvec

            pltpu.sync_copy(rows_v, acc_sh.at[dst_v.at[i]], add=True)

        plsc.subcore_barrier()
        pltpu.sync_copy(acc_sh.at[pl.ds(base, rows_per_tile)],
                        part_hbm.at[c].at[pl.ds(base, rows_per_tile)])

    return agg_kernel(x, src, dst, ew)


def _tc_finish(parts, W, bias, n_out, blk=1000):
    d = parts.shape[2]
    d_out = W.shape[1]

    def body(p_ref, w_ref, b_ref, o_ref):
        agg = p_ref[0] + p_ref[1]
        o_ref[...] = jnp.dot(agg, w_ref[...],
                             preferred_element_type=jnp.float32) + b_ref[...]

    return pl.pallas_call(
        body,
        grid=(n_out // blk,),
        in_specs=[
            pl.BlockSpec((NC, blk, d), lambda i: (0, i, 0)),
            pl.BlockSpec((d, d_out), lambda i: (0, 0)),
            pl.BlockSpec((1, d_out), lambda i: (0, 0)),
        ],
        out_specs=pl.BlockSpec((blk, d_out), lambda i: (i, 0)),
        out_shape=jax.ShapeDtypeStruct((n_out, d_out), jnp.float32),
    )(parts, W, bias.reshape(1, d_out))


def kernel(x, edge_index, edge_weight, W, bias):
    n, d = x.shape
    e = edge_weight.shape[0]
    nw = NC * NS

    e_pad = ((e + nw * GK - 1) // (nw * GK)) * nw * GK
    pad = e_pad - e
    n_chunks = e_pad // (nw * GK)
    dst = jnp.pad(edge_index[0], (0, pad)).reshape(nw, n_chunks, GK)
    src = jnp.pad(edge_index[1], (0, pad)).reshape(nw, n_chunks, GK)
    ew = jnp.pad(edge_weight, (0, pad)).reshape(nw, n_chunks, GK)

    n_pad = ((n + NS * 8 - 1) // (NS * 8)) * NS * 8
    parts = _sc_aggregate(x, src, dst, ew, n_pad, d, n_chunks)
    return _tc_finish(parts, W, bias, n)
